# baseline (device time: 11377 ns/iter reference)
import jax
import jax.numpy as jnp
from jax import lax
from jax.experimental import pallas as pl
from jax.experimental.pallas import tpu as pltpu

N_DEV = 16
EPS = 1e-5


def kernel(x, gamma, beta):
    m, n = x.shape
    n_global = n * N_DEV
    r = m // 128

    def body(x_ref, g_ref, b_ref, out_ref, comm_ref, send_sems, recv_sems):
        my = lax.axis_index("i")

        barrier_sem = pltpu.get_barrier_semaphore()
        for d in range(1, N_DEV):
            pl.semaphore_signal(
                barrier_sem, inc=1,
                device_id=((my + d) % N_DEV,),
                device_id_type=pl.DeviceIdType.MESH,
            )

        xv = x_ref[:, :]
        s1 = jnp.sum(xv, axis=1)
        s2 = jnp.sum(xv * xv, axis=1)
        comm_ref[0, 0:r, :] = jnp.reshape(s1, (r, 128))
        comm_ref[0, r:2 * r, :] = jnp.reshape(s2, (r, 128))
        pl.semaphore_wait(barrier_sem, N_DEV - 1)

        rdmas = []
        for d in range(1, N_DEV):
            rdma = pltpu.make_async_remote_copy(
                src_ref=comm_ref.at[0],
                dst_ref=comm_ref.at[d],
                send_sem=send_sems.at[d],
                recv_sem=recv_sems.at[d],
                device_id=((my + d) % N_DEV,),
                device_id_type=pl.DeviceIdType.MESH,
            )
            rdma.start()
            rdmas.append(rdma)

        g = g_ref[:][None, :]
        b = b_ref[:][None, :]
        xg = xv * g

        tot = comm_ref[0, :, :]
        for d, rdma in zip(range(1, N_DEV), rdmas):
            rdma.wait_recv()
            tot = tot + comm_ref[d, :, :]
        s1_l = jnp.reshape(tot[0:r, :], (m,))
        s2_l = jnp.reshape(tot[r:2 * r, :], (m,))
        mean_l = s1_l * (1.0 / n_global)
        ex2_l = s2_l * (1.0 / n_global)
        var_l = ex2_l - mean_l * mean_l
        inv_l = lax.rsqrt(var_l + EPS)

        mean_c = jnp.reshape(mean_l, (m, 1))
        inv_c = jnp.reshape(inv_l, (m, 1))
        out_ref[:, :] = xg * inv_c - g * (mean_c * inv_c) + b

        for rdma in rdmas:
            rdma.wait_send()

    return pl.pallas_call(
        body,
        out_shape=jax.ShapeDtypeStruct((m, n), jnp.float32),
        in_specs=[
            pl.BlockSpec(memory_space=pltpu.VMEM),
            pl.BlockSpec(memory_space=pltpu.VMEM),
            pl.BlockSpec(memory_space=pltpu.VMEM),
        ],
        out_specs=pl.BlockSpec(memory_space=pltpu.VMEM),
        scratch_shapes=[
            pltpu.VMEM((N_DEV, 2 * r, 128), jnp.float32),
            pltpu.SemaphoreType.DMA((N_DEV,)),
            pltpu.SemaphoreType.DMA((N_DEV,)),
        ],
        compiler_params=pltpu.CompilerParams(collective_id=0),
    )(x, gamma, beta)
